# Initial kernel scaffold; baseline (speedup 1.0000x reference)
import jax
import jax.numpy as jnp
from jax.experimental import pallas as pl


def kernel(points, features, knn_idx, W1, b1, W2, b2):
    raise NotImplementedError("write your pallas kernel here")


# speedup vs baseline: 2.9126x; 2.9126x over previous
import functools

import jax
import jax.numpy as jnp
from jax import lax
from jax.experimental import pallas as pl
from jax.experimental.pallas import tpu as pltpu
from jax.experimental.pallas import tpu_sc as plsc

_NC = 2
_NS = 16
_NW = _NC * _NS
_CH = 128


def _proj_body(f_ref, w_ref, b_ref, o_ref):
    o_ref[...] = (
        jnp.dot(f_ref[...], w_ref[...], preferred_element_type=jnp.float32)
        + b_ref[...]
    )


def _make_gather(ep, dp, dx, cpt):
    mesh = plsc.VectorSubcoreMesh(
        core_axis_name="c", subcore_axis_name="s",
        num_cores=_NC, num_subcores=_NS,
    )

    @functools.partial(
        pl.kernel,
        out_type=(
            jax.ShapeDtypeStruct((ep, dp), jnp.float32),
            jax.ShapeDtypeStruct((ep, dx), jnp.float32),
        ),
        mesh=mesh,
        scratch_types=[
            pltpu.VMEM((cpt, _CH), jnp.int32),
            pltpu.VMEM((_CH, dp), jnp.float32),
            pltpu.VMEM((_CH, dx), jnp.float32),
            pltpu.SemaphoreType.DMA,
            pltpu.SemaphoreType.DMA,
        ],
        compiler_params=pltpu.CompilerParams(use_tc_tiling_on_sc=False),
    )
    def gather_k(idx_hbm, proj_hbm, pts_hbm, gp_hbm, gx_hbm,
                 idx_v, bufp, bufx, semp, semx):
        wid = lax.axis_index("s") * _NC + lax.axis_index("c")
        row0 = wid * cpt
        pltpu.sync_copy(idx_hbm.at[pl.ds(row0, cpt)], idx_v)

        def body(i, carry):
            cp = pltpu.async_copy(proj_hbm.at[idx_v.at[i]], bufp, semp)
            cx = pltpu.async_copy(pts_hbm.at[idx_v.at[i]], bufx, semx)
            cp.wait()
            cx.wait()
            base = (row0 + i) * _CH
            pltpu.sync_copy(bufp, gp_hbm.at[pl.ds(base, _CH)])
            pltpu.sync_copy(bufx, gx_hbm.at[pl.ds(base, _CH)])
            return carry

        lax.fori_loop(0, cpt, body, 0)

    return gather_k


def _make_finish(blk, k_, dp, dx, dh):
    rpb = blk * k_

    def finish_body(gp_ref, gx_ref, pc_ref, w1p_ref, w1d_ref, b1_ref,
                    w2b_ref, o_ref):
        gx = gx_ref[...]
        center = pc_ref[...]
        rep = jnp.broadcast_to(
            center[:, None, :], (blk, k_, dx)).reshape(rpb, dx)
        diff = rep - gx
        ssq = jnp.sum(diff * diff, axis=1, keepdims=True)
        dist = jnp.sqrt(ssq + 1e-12)
        g1 = jnp.dot(diff, w1p_ref[...], preferred_element_type=jnp.float32)
        g1 = g1 + dist * w1d_ref[...] + b1_ref[...]
        g1 = jnp.where(g1 >= 0, g1, 0.2 * g1)
        z = jnp.dot(g1, w2b_ref[...], preferred_element_type=jnp.float32)
        z = z + gp_ref[...]
        z = jnp.where(z >= 0, z, 0.2 * z)
        o_ref[...] = jnp.mean(z.reshape(blk, k_, dh), axis=1)

    return finish_body


def kernel(points, features, knn_idx, W1, b1, W2, b2):
    b_, n_, _ = points.shape
    k_ = knn_idx.shape[1]
    d_ = features.shape[-1]
    dh = W2.shape[1]
    dx = 16
    e_ = n_ * k_

    pts = points.reshape(n_, 3)
    feats = features.reshape(n_, d_)

    pts_pad = jnp.zeros((n_, dx), jnp.float32).at[:, :3].set(pts)
    w2_top = W2[:d_]
    w2_bot = W2[d_:]
    w1_pad = jnp.zeros((dx, dh), jnp.float32).at[:3].set(W1[:3])
    w1_dist = W1[3:4]
    b1r = b1.reshape(1, dh)
    b2r = b2.reshape(1, dh)

    cpt = (e_ + _NW * _CH - 1) // (_NW * _CH)
    cpt = ((cpt + 7) // 8) * 8
    ep = cpt * _NW * _CH
    idx_flat = jnp.pad(knn_idx.reshape(-1), (0, ep - e_))
    idx2d = idx_flat.reshape(ep // _CH, _CH)

    proj = pl.pallas_call(
        _proj_body,
        out_shape=jax.ShapeDtypeStruct((n_, dh), jnp.float32),
    )(feats, w2_top, b2r)

    gp, gx = _make_gather(ep, dh, dx, cpt)(idx2d, proj, pts_pad)

    blk = 400
    nb = n_ // blk
    rpb = blk * k_
    out = pl.pallas_call(
        _make_finish(blk, k_, dh, dx, dh),
        grid=(nb,),
        in_specs=[
            pl.BlockSpec((rpb, dh), lambda i: (i, 0)),
            pl.BlockSpec((rpb, dx), lambda i: (i, 0)),
            pl.BlockSpec((blk, dx), lambda i: (i, 0)),
            pl.BlockSpec((dx, dh), lambda i: (0, 0)),
            pl.BlockSpec((1, dh), lambda i: (0, 0)),
            pl.BlockSpec((1, dh), lambda i: (0, 0)),
            pl.BlockSpec((dh, dh), lambda i: (0, 0)),
        ],
        out_specs=pl.BlockSpec((blk, dh), lambda i: (i, 0)),
        out_shape=jax.ShapeDtypeStruct((n_, dh), jnp.float32),
    )(gp, gx, pts_pad, w1_pad, w1_dist, b1r, w2_bot)

    return out.reshape(b_, n_, dh)


# speedup vs baseline: 3.2055x; 1.1006x over previous
import functools

import jax
import jax.numpy as jnp
from jax import lax
from jax.experimental import pallas as pl
from jax.experimental.pallas import tpu as pltpu
from jax.experimental.pallas import tpu_sc as plsc

_NC = 2
_NS = 16
_NW = _NC * _NS
_CH = 128


def _proj_body(f_ref, w_ref, b_ref, o_ref):
    o_ref[...] = (
        jnp.dot(f_ref[...], w_ref[...], preferred_element_type=jnp.float32)
        + b_ref[...]
    )


def _make_gather(ep, dp, dx, cpt):
    mesh = plsc.VectorSubcoreMesh(
        core_axis_name="c", subcore_axis_name="s",
        num_cores=_NC, num_subcores=_NS,
    )

    nbuf = 4
    ngroups = cpt // nbuf

    @functools.partial(
        pl.kernel,
        out_type=(
            jax.ShapeDtypeStruct((ep, dp), jnp.float32),
            jax.ShapeDtypeStruct((ep, dx), jnp.float32),
        ),
        mesh=mesh,
        scratch_types=[
            pltpu.VMEM((cpt, _CH), jnp.int32),
            pltpu.VMEM((2, nbuf, _CH, dp), jnp.float32),
            pltpu.VMEM((2, nbuf, _CH, dx), jnp.float32),
            pltpu.SemaphoreType.DMA,
            pltpu.SemaphoreType.DMA,
        ],
        compiler_params=pltpu.CompilerParams(use_tc_tiling_on_sc=False),
    )
    def gather_k(idx_hbm, proj_hbm, pts_hbm, gp_hbm, gx_hbm,
                 idx_v, bufp, bufx, sem0, sem1):
        wid = lax.axis_index("s") * _NC + lax.axis_index("c")
        row0 = wid * cpt
        pltpu.sync_copy(idx_hbm.at[pl.ds(row0, cpt)], idx_v)
        sems = (sem0, sem1)

        def fire(g, h):
            for b in range(nbuf):
                i = g * nbuf + b
                pltpu.async_copy(proj_hbm.at[idx_v.at[i]], bufp.at[h, b],
                                 sems[h])
                pltpu.async_copy(pts_hbm.at[idx_v.at[i]], bufx.at[h, b],
                                 sems[h])

        def drain_and_store(g, h):
            for b in range(nbuf):
                pltpu.make_async_copy(
                    proj_hbm.at[pl.ds(0, _CH)], bufp.at[h, b], sems[h]).wait()
                pltpu.make_async_copy(
                    pts_hbm.at[pl.ds(0, _CH)], bufx.at[h, b], sems[h]).wait()
            for b in range(nbuf):
                base = (row0 + (g * nbuf + b)) * _CH
                pltpu.sync_copy(bufp.at[h, b], gp_hbm.at[pl.ds(base, _CH)])
                pltpu.sync_copy(bufx.at[h, b], gx_hbm.at[pl.ds(base, _CH)])

        fire(0, 0)

        def body(t, carry):
            g0 = 2 * t
            fire(g0 + 1, 1)
            drain_and_store(g0, 0)

            @pl.when(t < ngroups // 2 - 1)
            def _():
                fire(g0 + 2, 0)

            drain_and_store(g0 + 1, 1)
            return carry

        lax.fori_loop(0, ngroups // 2, body, 0)

    return gather_k


def _make_finish(blk, k_, dp, dx, dh):
    rpb = blk * k_

    def finish_body(gp_ref, gx_ref, pc_ref, w1p_ref, w1d_ref, b1_ref,
                    w2b_ref, o_ref):
        gx = gx_ref[...]
        center = pc_ref[...]
        rep = jnp.broadcast_to(
            center[:, None, :], (blk, k_, dx)).reshape(rpb, dx)
        diff = rep - gx
        ssq = jnp.sum(diff * diff, axis=1, keepdims=True)
        dist = jnp.sqrt(ssq + 1e-12)
        g1 = jnp.dot(diff, w1p_ref[...], preferred_element_type=jnp.float32)
        g1 = g1 + dist * w1d_ref[...] + b1_ref[...]
        g1 = jnp.where(g1 >= 0, g1, 0.2 * g1)
        z = jnp.dot(g1, w2b_ref[...], preferred_element_type=jnp.float32)
        z = z + gp_ref[...]
        z = jnp.where(z >= 0, z, 0.2 * z)
        o_ref[...] = jnp.mean(z.reshape(blk, k_, dh), axis=1)

    return finish_body


def kernel(points, features, knn_idx, W1, b1, W2, b2):
    b_, n_, _ = points.shape
    k_ = knn_idx.shape[1]
    d_ = features.shape[-1]
    dh = W2.shape[1]
    dx = 16
    e_ = n_ * k_

    pts = points.reshape(n_, 3)
    feats = features.reshape(n_, d_)

    pts_pad = jnp.zeros((n_, dx), jnp.float32).at[:, :3].set(pts)
    w2_top = W2[:d_]
    w2_bot = W2[d_:]
    w1_pad = jnp.zeros((dx, dh), jnp.float32).at[:3].set(W1[:3])
    w1_dist = W1[3:4]
    b1r = b1.reshape(1, dh)
    b2r = b2.reshape(1, dh)

    cpt = (e_ + _NW * _CH - 1) // (_NW * _CH)
    cpt = ((cpt + 7) // 8) * 8
    ep = cpt * _NW * _CH
    idx_flat = jnp.pad(knn_idx.reshape(-1), (0, ep - e_))
    idx2d = idx_flat.reshape(ep // _CH, _CH)

    proj = pl.pallas_call(
        _proj_body,
        out_shape=jax.ShapeDtypeStruct((n_, dh), jnp.float32),
    )(feats, w2_top, b2r)

    gp, gx = _make_gather(ep, dh, dx, cpt)(idx2d, proj, pts_pad)

    blk = 400
    nb = n_ // blk
    rpb = blk * k_
    out = pl.pallas_call(
        _make_finish(blk, k_, dh, dx, dh),
        grid=(nb,),
        in_specs=[
            pl.BlockSpec((rpb, dh), lambda i: (i, 0)),
            pl.BlockSpec((rpb, dx), lambda i: (i, 0)),
            pl.BlockSpec((blk, dx), lambda i: (i, 0)),
            pl.BlockSpec((dx, dh), lambda i: (0, 0)),
            pl.BlockSpec((1, dh), lambda i: (0, 0)),
            pl.BlockSpec((1, dh), lambda i: (0, 0)),
            pl.BlockSpec((dh, dh), lambda i: (0, 0)),
        ],
        out_specs=pl.BlockSpec((blk, dh), lambda i: (i, 0)),
        out_shape=jax.ShapeDtypeStruct((n_, dh), jnp.float32),
    )(gp, gx, pts_pad, w1_pad, w1_dist, b1r, w2_bot)

    return out.reshape(b_, n_, dh)


# speedup vs baseline: 3.4576x; 1.0786x over previous
import functools

import jax
import jax.numpy as jnp
from jax import lax
from jax.experimental import pallas as pl
from jax.experimental.pallas import tpu as pltpu
from jax.experimental.pallas import tpu_sc as plsc

_NC = 2
_NS = 16
_NW = _NC * _NS
_CH = 128


def _proj_body(f_ref, w_ref, b_ref, o_ref):
    o_ref[...] = (
        jnp.dot(f_ref[...], w_ref[...], preferred_element_type=jnp.float32)
        + b_ref[...]
    ).astype(jnp.bfloat16)


def _make_gather(ep, dp, dx, cpt):
    mesh = plsc.VectorSubcoreMesh(
        core_axis_name="c", subcore_axis_name="s",
        num_cores=_NC, num_subcores=_NS,
    )

    nbuf = 4
    ngroups = cpt // nbuf

    @functools.partial(
        pl.kernel,
        out_type=(
            jax.ShapeDtypeStruct((ep, dp), jnp.bfloat16),
            jax.ShapeDtypeStruct((ep, dx), jnp.float32),
        ),
        mesh=mesh,
        scratch_types=[
            pltpu.VMEM((cpt, _CH), jnp.int32),
            pltpu.VMEM((2, nbuf, _CH, dp), jnp.bfloat16),
            pltpu.VMEM((2, nbuf, _CH, dx), jnp.float32),
            pltpu.SemaphoreType.DMA,
            pltpu.SemaphoreType.DMA,
        ],
        compiler_params=pltpu.CompilerParams(use_tc_tiling_on_sc=False),
    )
    def gather_k(idx_hbm, proj_hbm, pts_hbm, gp_hbm, gx_hbm,
                 idx_v, bufp, bufx, sem0, sem1):
        wid = lax.axis_index("s") * _NC + lax.axis_index("c")
        row0 = wid * cpt
        pltpu.sync_copy(idx_hbm.at[pl.ds(row0, cpt)], idx_v)
        sems = (sem0, sem1)

        def fire(g, h):
            for b in range(nbuf):
                i = g * nbuf + b
                pltpu.async_copy(proj_hbm.at[idx_v.at[i]], bufp.at[h, b],
                                 sems[h])
                pltpu.async_copy(pts_hbm.at[idx_v.at[i]], bufx.at[h, b],
                                 sems[h])

        def drain_and_store(g, h):
            for b in range(nbuf):
                pltpu.make_async_copy(
                    proj_hbm.at[pl.ds(0, _CH)], bufp.at[h, b], sems[h]).wait()
                pltpu.make_async_copy(
                    pts_hbm.at[pl.ds(0, _CH)], bufx.at[h, b], sems[h]).wait()
            for b in range(nbuf):
                base = (row0 + (g * nbuf + b)) * _CH
                pltpu.sync_copy(bufp.at[h, b], gp_hbm.at[pl.ds(base, _CH)])
                pltpu.sync_copy(bufx.at[h, b], gx_hbm.at[pl.ds(base, _CH)])

        fire(0, 0)

        def body(t, carry):
            g0 = 2 * t
            fire(g0 + 1, 1)
            drain_and_store(g0, 0)

            @pl.when(t < ngroups // 2 - 1)
            def _():
                fire(g0 + 2, 0)

            drain_and_store(g0 + 1, 1)
            return carry

        lax.fori_loop(0, ngroups // 2, body, 0)

    return gather_k


def _make_finish(blk, k_, dp, dx, dh):
    rpb = blk * k_

    def finish_body(gp_ref, gx_ref, pc_ref, w1p_ref, w1d_ref, b1_ref,
                    w2b_ref, o_ref):
        gx = gx_ref[...]
        center = pc_ref[...]
        rep = jnp.broadcast_to(
            center[:, None, :], (blk, k_, dx)).reshape(rpb, dx)
        diff = rep - gx
        ssq = jnp.sum(diff * diff, axis=1, keepdims=True)
        dist = jnp.sqrt(ssq + 1e-12)
        g1 = jnp.dot(diff, w1p_ref[...], preferred_element_type=jnp.float32)
        g1 = g1 + dist * w1d_ref[...] + b1_ref[...]
        g1 = jnp.where(g1 >= 0, g1, 0.2 * g1)
        z = jnp.dot(g1, w2b_ref[...], preferred_element_type=jnp.float32)
        z = z + gp_ref[...].astype(jnp.float32)
        z = jnp.where(z >= 0, z, 0.2 * z)
        o_ref[...] = jnp.mean(z.reshape(blk, k_, dh), axis=1)

    return finish_body


def kernel(points, features, knn_idx, W1, b1, W2, b2):
    b_, n_, _ = points.shape
    k_ = knn_idx.shape[1]
    d_ = features.shape[-1]
    dh = W2.shape[1]
    dx = 16
    e_ = n_ * k_

    pts = points.reshape(n_, 3)
    feats = features.reshape(n_, d_)

    pts_pad = jnp.zeros((n_, dx), jnp.float32).at[:, :3].set(pts)
    w2_top = W2[:d_]
    w2_bot = W2[d_:]
    w1_pad = jnp.zeros((dx, dh), jnp.float32).at[:3].set(W1[:3])
    w1_dist = W1[3:4]
    b1r = b1.reshape(1, dh)
    b2r = b2.reshape(1, dh)

    cpt = (e_ + _NW * _CH - 1) // (_NW * _CH)
    cpt = ((cpt + 7) // 8) * 8
    ep = cpt * _NW * _CH
    idx_flat = jnp.pad(knn_idx.reshape(-1), (0, ep - e_))
    idx2d = idx_flat.reshape(ep // _CH, _CH)

    proj = pl.pallas_call(
        _proj_body,
        out_shape=jax.ShapeDtypeStruct((n_, dh), jnp.bfloat16),
    )(feats, w2_top, b2r)

    gp, gx = _make_gather(ep, dh, dx, cpt)(idx2d, proj, pts_pad)

    blk = 400
    nb = n_ // blk
    rpb = blk * k_
    out = pl.pallas_call(
        _make_finish(blk, k_, dh, dx, dh),
        grid=(nb,),
        in_specs=[
            pl.BlockSpec((rpb, dh), lambda i: (i, 0)),
            pl.BlockSpec((rpb, dx), lambda i: (i, 0)),
            pl.BlockSpec((blk, dx), lambda i: (i, 0)),
            pl.BlockSpec((dx, dh), lambda i: (0, 0)),
            pl.BlockSpec((1, dh), lambda i: (0, 0)),
            pl.BlockSpec((1, dh), lambda i: (0, 0)),
            pl.BlockSpec((dh, dh), lambda i: (0, 0)),
        ],
        out_specs=pl.BlockSpec((blk, dh), lambda i: (i, 0)),
        out_shape=jax.ShapeDtypeStruct((n_, dh), jnp.float32),
    )(gp, gx, pts_pad, w1_pad, w1_dist, b1r, w2_bot)

    return out.reshape(b_, n_, dh)
